# baseline (speedup 1.0000x reference)
import functools

import jax
import jax.numpy as jnp
from jax import lax
from jax.experimental import pallas as pl
from jax.experimental.pallas import tpu as pltpu
from jax.experimental.pallas import tpu_sc as plsc

BATCH = 16384
HIST = 50
EMBED_DIM = 32
QROWS = 250000

NUM_CORES = 2
NUM_SUBCORES = 16
NW = NUM_CORES * NUM_SUBCORES
NBT = BATCH // 128
NHT = (HIST + 7) // 8
TILES = NBT * NHT
TPW = TILES // NW


def _gather_kernel(idx_t, table_c, out_t, idx_v, qbuf, rows_v, outb,
                   gsem0, gsem1, wsem0, wsem1, isem):
    wid = lax.axis_index("s") * NUM_CORES + lax.axis_index("c")
    lane = lax.broadcasted_iota(jnp.int32, (16,), 0)
    gsems = (gsem0, gsem1)
    wsems = (wsem0, wsem1)

    def load_tile(h_hi, b_hi):
        pltpu.async_copy(
            idx_t.at[pl.ds(h_hi * 8, 8), pl.ds(b_hi * 128, 128)],
            idx_v, isem).wait()

    def compute_q(r):
        for g in range(8):
            iv = idx_v[r, pl.ds(g * 16, 16)]
            qbuf[r, pl.ds(g * 16, 16)] = jnp.right_shift(iv, 2)

    def start_gather(r):
        pltpu.async_copy(table_c.at[qbuf.at[r]], rows_v.at[r % 2],
                         gsems[r % 2])

    def wait_gather(r):
        pltpu.make_async_copy(table_c.at[qbuf.at[r]], rows_v.at[r % 2],
                              gsems[r % 2]).wait()

    def transpose_row(r):
        buf = r % 2
        src0 = []
        slots = [lane + g * 16 for g in range(8)]
        for g in range(8):
            iv = idx_v[r, pl.ds(g * 16, 16)]
            src0.append(jnp.bitwise_and(iv, 3) * 32)
        rbuf = rows_v.at[buf]
        obuf = outb.at[buf]
        zero = jnp.zeros((16,), jnp.int32)

        def d_body(d, carry):
            srcs, dvec = carry
            nsrcs = []
            for g in range(8):
                vals = plsc.load_gather(rbuf, [slots[g], srcs[g]])
                plsc.store_scatter(obuf, [dvec, slots[g]], vals)
                nsrcs.append(srcs[g] + 1)
            return tuple(nsrcs), dvec + 1

        lax.fori_loop(0, EMBED_DIM, d_body, (tuple(src0), zero))

    def start_writes(r, h_hi, b_hi):
        buf = r % 2
        h = h_hi * 8 + r
        for d_hi in range(4):
            pltpu.async_copy(
                outb.at[buf, pl.ds(d_hi * 8, 8), :],
                out_t.at[h, pl.ds(d_hi * 8, 8), pl.ds(b_hi * 128, 128)],
                wsems[buf])

    def wait_writes(r, h_hi, b_hi):
        buf = r % 2
        h = h_hi * 8 + r
        for d_hi in range(4):
            pltpu.make_async_copy(
                outb.at[buf, pl.ds(d_hi * 8, 8), :],
                out_t.at[h, pl.ds(d_hi * 8, 8), pl.ds(b_hi * 128, 128)],
                wsems[buf]).wait()

    def do_tile(h_hi, b_hi, nrows):
        load_tile(h_hi, b_hi)
        for r in range(nrows):
            compute_q(r)
        start_gather(0)
        if nrows > 1:
            start_gather(1)
        for r in range(nrows):
            wait_gather(r)
            if r >= 2:
                wait_writes(r - 2, h_hi, b_hi)
            transpose_row(r)
            if r + 2 < nrows:
                start_gather(r + 2)
            start_writes(r, h_hi, b_hi)
        for r in range(max(nrows - 2, 0), nrows):
            wait_writes(r, h_hi, b_hi)

    def full_body(ip, c):
        i = 7 * (ip // 6) + ip % 6
        t = wid * TPW + i
        do_tile(t % NHT, t // NHT, 8)
        return c

    lax.fori_loop(0, 24, full_body, 0)

    def part_body(a, c):
        t = wid * TPW + 7 * a + 6
        do_tile(t % NHT, t // NHT, 2)
        return c

    lax.fori_loop(0, 4, part_body, 0)


VTILES = 7812
CPW = 246


def _repack_kernel(table_t, table_c, nat, outv, lsem, wsem0, wsem1):
    wid = lax.axis_index("s") * NUM_CORES + lax.axis_index("c")
    lane = lax.broadcasted_iota(jnp.int32, (16,), 0)
    dvecs, vbases = [], []
    for m in range(8):
        c = lane + m * 16
        dvecs.append(jnp.bitwise_and(c, 31))
        vbases.append(jnp.right_shift(c, 5))

    wsems = (wsem0, wsem1)
    base_col = wid * CPW

    def do_col(col, buf):
        pltpu.async_copy(
            table_t.at[:, pl.ds(col * 128, 128)], nat.at[buf], lsem).wait()

        @pl.when(col >= base_col + 2)
        def _w():
            pltpu.make_async_copy(
                outv.at[buf], table_c.at[pl.ds((col - 2) * 32, 32), :],
                wsems[buf]).wait()

        def j_body(j, jc):
            for m in range(8):
                vals = plsc.load_gather(nat.at[buf],
                                        [dvecs[m], vbases[m] + 4 * j])
                outv[buf, j, pl.ds(m * 16, 16)] = vals
            return jc

        lax.fori_loop(0, 32, j_body, 0)
        pltpu.async_copy(outv.at[buf],
                         table_c.at[pl.ds(col * 32, 32), :], wsems[buf])

    def pair_body(ii, carry):
        for k in range(2):
            col = base_col + 2 * ii + k

            @pl.when(col < VTILES)
            def _p(col=col, k=k):
                do_col(col, k)

        return carry

    lax.fori_loop(0, CPW // 2, pair_body, 0)
    nvalid = jnp.minimum(CPW, VTILES - base_col)
    for p in range(2):
        m_last = nvalid - 1
        last = m_last - jnp.bitwise_and(m_last - p, 1)

        @pl.when(last >= 0)
        def _d(last=last, p=p):
            pltpu.make_async_copy(
                outv.at[p],
                table_c.at[pl.ds((base_col + last) * 32, 32), :],
                wsems[p]).wait()

    @pl.when(wid == 0)
    def _tail():
        tail_off = pl.multiple_of(VTILES * 128 + wid * 0, 128)
        pltpu.async_copy(
            table_t.at[:, pl.ds(tail_off, 128)], nat.at[0], lsem).wait()

        def j_body(j, jc):
            for m in range(8):
                vals = plsc.load_gather(nat.at[0],
                                        [dvecs[m], vbases[m] + 4 * j])
                outv[0, j, pl.ds(m * 16, 16)] = vals
            return jc

        lax.fori_loop(0, 16, j_body, 0)
        pltpu.async_copy(outv.at[0, pl.ds(0, 16), :],
                         table_c.at[pl.ds(VTILES * 32, 16), :], wsem0).wait()


@jax.jit
def _repack(table_t):
    mesh = plsc.VectorSubcoreMesh(core_axis_name="c", subcore_axis_name="s")
    f = functools.partial(
        pl.kernel,
        mesh=mesh,
        out_type=jax.ShapeDtypeStruct((QROWS, 128), jnp.float32),
        scratch_types=[
            pltpu.VMEM((2, 32, 128), jnp.float32),
            pltpu.VMEM((2, 32, 128), jnp.float32),
            pltpu.SemaphoreType.DMA,
            pltpu.SemaphoreType.DMA,
            pltpu.SemaphoreType.DMA,
        ],
        compiler_params=pltpu.CompilerParams(use_tc_tiling_on_sc=True,
                                             needs_layout_passes=False),
    )(_repack_kernel)
    return f(table_t)


@jax.jit
def _run(idx_t, table_c):
    mesh = plsc.VectorSubcoreMesh(core_axis_name="c", subcore_axis_name="s")
    f = functools.partial(
        pl.kernel,
        mesh=mesh,
        out_type=jax.ShapeDtypeStruct((HIST, EMBED_DIM, BATCH), jnp.float32),
        scratch_types=[
            pltpu.VMEM((8, 128), jnp.int32),
            pltpu.VMEM((8, 128), jnp.int32),
            pltpu.VMEM((2, 128, 128), jnp.float32),
            pltpu.VMEM((2, EMBED_DIM, 128), jnp.float32),
            pltpu.SemaphoreType.DMA,
            pltpu.SemaphoreType.DMA,
            pltpu.SemaphoreType.DMA,
            pltpu.SemaphoreType.DMA,
            pltpu.SemaphoreType.DMA,
        ],
        compiler_params=pltpu.CompilerParams(use_tc_tiling_on_sc=True,
                                             needs_layout_passes=False),
    )(_gather_kernel)
    return f(idx_t, table_c)


def kernel(inputs, table):
    idx_t = inputs.astype(jnp.int32).T
    table_c = _repack(table.T)
    out_t = _run(idx_t, table_c)
    return out_t.transpose(2, 0, 1)


# speedup vs baseline: 1.3877x; 1.3877x over previous
import functools

import jax
import jax.numpy as jnp
from jax import lax
from jax.experimental import pallas as pl
from jax.experimental.pallas import tpu as pltpu
from jax.experimental.pallas import tpu_sc as plsc

BATCH = 16384
HIST = 50
EMBED_DIM = 32
QROWS = 250000

NUM_CORES = 2
NUM_SUBCORES = 16
NW = NUM_CORES * NUM_SUBCORES
NBT = BATCH // 128
NHT = (HIST + 7) // 8
TILES = NBT * NHT
TPW = TILES // NW


def _gather_kernel(idx_t, table_c, out_t, idx_v, qbuf, rows_v, outb,
                   gsem0, gsem1, wsem0, wsem1, isem):
    wid = lax.axis_index("s") * NUM_CORES + lax.axis_index("c")
    lane = lax.broadcasted_iota(jnp.int32, (16,), 0)
    gsems = (gsem0, gsem1)
    wsems = (wsem0, wsem1)

    def load_tile(h_hi, b_hi):
        pltpu.async_copy(
            idx_t.at[pl.ds(h_hi * 8, 8), pl.ds(b_hi * 128, 128)],
            idx_v, isem).wait()

    def compute_q(r):
        for g in range(8):
            iv = idx_v[r, pl.ds(g * 16, 16)]
            qbuf[r, pl.ds(g * 16, 16)] = jnp.right_shift(iv, 2)

    def start_gather(r):
        pltpu.async_copy(table_c.at[qbuf.at[r]], rows_v.at[r % 2],
                         gsems[r % 2])

    def wait_gather(r):
        pltpu.make_async_copy(table_c.at[qbuf.at[r]], rows_v.at[r % 2],
                              gsems[r % 2]).wait()

    def transpose_row(r):
        buf = r % 2
        src0 = []
        slots = [lane + g * 16 for g in range(8)]
        for g in range(8):
            iv = idx_v[r, pl.ds(g * 16, 16)]
            src0.append(jnp.bitwise_and(iv, 3) * 32)
        rbuf = rows_v.at[buf]
        obuf = outb.at[buf]
        zero = jnp.zeros((16,), jnp.int32)

        def d_body(i, carry):
            srcs, dvec = carry
            for k in range(4):
                for g in range(8):
                    vals = plsc.load_gather(rbuf, [slots[g], srcs[g] + k])
                    plsc.store_scatter(obuf, [dvec + k, slots[g]], vals)
            return tuple(s + 4 for s in srcs), dvec + 4

        lax.fori_loop(0, EMBED_DIM // 4, d_body, (tuple(src0), zero))

    def start_writes(r, h_hi, b_hi):
        buf = r % 2
        h = h_hi * 8 + r
        for d_hi in range(4):
            pltpu.async_copy(
                outb.at[buf, pl.ds(d_hi * 8, 8), :],
                out_t.at[h, pl.ds(d_hi * 8, 8), pl.ds(b_hi * 128, 128)],
                wsems[buf])

    def wait_writes(r, h_hi, b_hi):
        buf = r % 2
        h = h_hi * 8 + r
        for d_hi in range(4):
            pltpu.make_async_copy(
                outb.at[buf, pl.ds(d_hi * 8, 8), :],
                out_t.at[h, pl.ds(d_hi * 8, 8), pl.ds(b_hi * 128, 128)],
                wsems[buf]).wait()

    def do_tile(h_hi, b_hi, nrows):
        load_tile(h_hi, b_hi)
        for r in range(nrows):
            compute_q(r)
        start_gather(0)
        if nrows > 1:
            start_gather(1)
        for r in range(nrows):
            wait_gather(r)
            if r >= 2:
                wait_writes(r - 2, h_hi, b_hi)
            transpose_row(r)
            if r + 2 < nrows:
                start_gather(r + 2)
            start_writes(r, h_hi, b_hi)
        for r in range(max(nrows - 2, 0), nrows):
            wait_writes(r, h_hi, b_hi)

    def full_body(ip, c):
        i = 7 * (ip // 6) + ip % 6
        t = wid * TPW + i
        do_tile(t % NHT, t // NHT, 8)
        return c

    lax.fori_loop(0, 24, full_body, 0)

    def part_body(a, c):
        t = wid * TPW + 7 * a + 6
        do_tile(t % NHT, t // NHT, 2)
        return c

    lax.fori_loop(0, 4, part_body, 0)


@jax.jit
def _run(idx_t, table_c):
    mesh = plsc.VectorSubcoreMesh(core_axis_name="c", subcore_axis_name="s")
    f = functools.partial(
        pl.kernel,
        mesh=mesh,
        out_type=jax.ShapeDtypeStruct((HIST, EMBED_DIM, BATCH), jnp.float32),
        scratch_types=[
            pltpu.VMEM((8, 128), jnp.int32),
            pltpu.VMEM((8, 128), jnp.int32),
            pltpu.VMEM((2, 128, 128), jnp.float32),
            pltpu.VMEM((2, EMBED_DIM, 128), jnp.float32),
            pltpu.SemaphoreType.DMA,
            pltpu.SemaphoreType.DMA,
            pltpu.SemaphoreType.DMA,
            pltpu.SemaphoreType.DMA,
            pltpu.SemaphoreType.DMA,
        ],
        compiler_params=pltpu.CompilerParams(use_tc_tiling_on_sc=True,
                                             needs_layout_passes=False),
    )(_gather_kernel)
    return f(idx_t, table_c)


def kernel(inputs, table):
    idx_t = inputs.astype(jnp.int32).T
    table_c = table.reshape(QROWS, 128)
    out_t = _run(idx_t, table_c)
    return out_t.transpose(2, 0, 1)


# speedup vs baseline: 1.3923x; 1.0033x over previous
import functools

import jax
import jax.numpy as jnp
from jax import lax
from jax.experimental import pallas as pl
from jax.experimental.pallas import tpu as pltpu
from jax.experimental.pallas import tpu_sc as plsc

BATCH = 16384
HIST = 50
EMBED_DIM = 32
QROWS = 250000

NUM_CORES = 2
NUM_SUBCORES = 16
NW = NUM_CORES * NUM_SUBCORES
NBT = BATCH // 128
NHT = (HIST + 7) // 8
TILES = NBT * NHT
TPW = TILES // NW


def _gather_kernel(idx_t, table_c, out_t, idx_v, qbuf, rows_v, outb,
                   gsem0, gsem1, wsem0, wsem1, isem):
    wid = lax.axis_index("s") * NUM_CORES + lax.axis_index("c")
    lane = lax.broadcasted_iota(jnp.int32, (16,), 0)
    gsems = (gsem0, gsem1)
    wsems = (wsem0, wsem1)

    def load_tile(h_hi, b_hi):
        pltpu.async_copy(
            idx_t.at[pl.ds(h_hi * 8, 8), pl.ds(b_hi * 128, 128)],
            idx_v, isem).wait()

    def compute_q(r):
        for g in range(8):
            iv = idx_v[r, pl.ds(g * 16, 16)]
            qbuf[r, pl.ds(g * 16, 16)] = jnp.right_shift(iv, 2)

    def start_gather(r):
        pltpu.async_copy(table_c.at[qbuf.at[r]], rows_v.at[r % 2],
                         gsems[r % 2])

    def wait_gather(r):
        pltpu.make_async_copy(table_c.at[qbuf.at[r]], rows_v.at[r % 2],
                              gsems[r % 2]).wait()

    def transpose_row(r):
        buf = r % 2
        src0 = []
        slots = [lane + g * 16 for g in range(8)]
        for g in range(8):
            iv = idx_v[r, pl.ds(g * 16, 16)]
            src0.append(jnp.bitwise_and(iv, 3) * 32)
        rbuf = rows_v.at[buf]
        obuf = outb.at[buf]
        zero = jnp.zeros((16,), jnp.int32)

        def d_body(d, carry):
            srcs, dvec = carry
            nsrcs = []
            for g in range(8):
                vals = plsc.load_gather(rbuf, [slots[g], srcs[g]])
                plsc.store_scatter(obuf, [dvec, slots[g]], vals)
                nsrcs.append(srcs[g] + 1)
            return tuple(nsrcs), dvec + 1

        lax.fori_loop(0, EMBED_DIM, d_body, (tuple(src0), zero))

    def start_writes(r, h_hi, b_hi):
        buf = r % 2
        h = h_hi * 8 + r
        for d_hi in range(4):
            pltpu.async_copy(
                outb.at[buf, pl.ds(d_hi * 8, 8), :],
                out_t.at[h, pl.ds(d_hi * 8, 8), pl.ds(b_hi * 128, 128)],
                wsems[buf])

    def wait_writes(r, h_hi, b_hi):
        buf = r % 2
        h = h_hi * 8 + r
        for d_hi in range(4):
            pltpu.make_async_copy(
                outb.at[buf, pl.ds(d_hi * 8, 8), :],
                out_t.at[h, pl.ds(d_hi * 8, 8), pl.ds(b_hi * 128, 128)],
                wsems[buf]).wait()

    def do_tile(h_hi, b_hi, nrows):
        load_tile(h_hi, b_hi)
        for r in range(nrows):
            compute_q(r)
        start_gather(0)
        if nrows > 1:
            start_gather(1)
        for r in range(nrows):
            wait_gather(r)
            if r >= 2:
                wait_writes(r - 2, h_hi, b_hi)
            transpose_row(r)
            if r + 2 < nrows:
                start_gather(r + 2)
            start_writes(r, h_hi, b_hi)
        for r in range(max(nrows - 2, 0), nrows):
            wait_writes(r, h_hi, b_hi)

    def full_body(ip, c):
        i = 7 * (ip // 6) + ip % 6
        t = wid * TPW + i
        do_tile(t % NHT, t // NHT, 8)
        return c

    lax.fori_loop(0, 24, full_body, 0)

    def part_body(a, c):
        t = wid * TPW + 7 * a + 6
        do_tile(t % NHT, t // NHT, 2)
        return c

    lax.fori_loop(0, 4, part_body, 0)


@jax.jit
def _run(idx_t, table_c):
    mesh = plsc.VectorSubcoreMesh(core_axis_name="c", subcore_axis_name="s")
    f = functools.partial(
        pl.kernel,
        mesh=mesh,
        out_type=jax.ShapeDtypeStruct((HIST, EMBED_DIM, BATCH), jnp.float32),
        scratch_types=[
            pltpu.VMEM((8, 128), jnp.int32),
            pltpu.VMEM((8, 128), jnp.int32),
            pltpu.VMEM((2, 128, 128), jnp.float32),
            pltpu.VMEM((2, EMBED_DIM, 128), jnp.float32),
            pltpu.SemaphoreType.DMA,
            pltpu.SemaphoreType.DMA,
            pltpu.SemaphoreType.DMA,
            pltpu.SemaphoreType.DMA,
            pltpu.SemaphoreType.DMA,
        ],
        compiler_params=pltpu.CompilerParams(use_tc_tiling_on_sc=True,
                                             needs_layout_passes=False),
    )(_gather_kernel)
    return f(idx_t, table_c)


def kernel(inputs, table):
    idx_t = inputs.astype(jnp.int32).T
    table_c = table.reshape(QROWS, 128)
    out_t = _run(idx_t, table_c)
    return out_t.transpose(2, 0, 1)
